# baseline (speedup 1.0000x reference)
import functools

import jax
import jax.numpy as jnp
from jax import lax
from jax.experimental import pallas as pl
from jax.experimental.pallas import tpu as pltpu
from jax.experimental.pallas import tpu_sc as plsc

_D = 64
_B = 4096
_QT = 20
_DT = 200
_NC = 2
_NS = 16
_NW = _NC * _NS
_CH = 256
_CPR = _B // _CH

_mesh = plsc.VectorSubcoreMesh(
    core_axis_name="c", subcore_axis_name="s", num_cores=_NC, num_subcores=_NS
)


def _make_gather(total_t, base_t, num_t):
    nchunks = num_t * _CPR
    cpw = nchunks // _NW
    span = (cpw - 1) // _CPR + 2
    span = min(span, num_t)

    @functools.partial(
        pl.kernel,
        out_type=jax.ShapeDtypeStruct((num_t * _B, _D), jnp.float32),
        mesh=_mesh,
        compiler_params=pltpu.CompilerParams(
            use_tc_tiling_on_sc=False, needs_layout_passes=False),
        scratch_types=[
            pltpu.VMEM((span, _B), jnp.int32),
            pltpu.VMEM((_CH, _D), jnp.float32),
            pltpu.VMEM((_CH, _D), jnp.float32),
            pltpu.SemaphoreType.DMA,
            pltpu.SemaphoreType.DMA,
            pltpu.SemaphoreType.DMA,
            pltpu.SemaphoreType.DMA,
        ],
    )
    def gather(tok_hbm, table_hbm, out_hbm, idx, r0, r1, gs0, gs1, ws0, ws1):
        w = lax.axis_index("s") * _NC + lax.axis_index("c")
        j0 = w * cpw
        t0 = jnp.minimum(j0 // _CPR, num_t - span)

        pltpu.sync_copy(tok_hbm.at[pl.ds(base_t + t0, span)], idx)

        def shift(i, carry):
            r = i // (_B // 16)
            o = (i - r * (_B // 16)) * 16
            idx[r, pl.ds(o, 16)] = idx[r, pl.ds(o, 16)] + 1
            return carry
        lax.fori_loop(0, span * (_B // 16), shift, 0)

        def fire_gather(j, rbuf, sem):
            t = j // _CPR
            off = (j - t * _CPR) * _CH
            lr = t - t0
            pltpu.async_copy(
                table_hbm.at[idx.at[lr, pl.ds(off, 128)]],
                rbuf.at[pl.ds(0, 128)], sem)
            pltpu.async_copy(
                table_hbm.at[idx.at[lr, pl.ds(off + 128, 128)]],
                rbuf.at[pl.ds(128, 128)], sem)

        def drain_gather(rbuf, sem):
            pltpu.make_async_copy(
                table_hbm.at[pl.ds(0, 128)], rbuf.at[pl.ds(0, 128)], sem).wait()
            pltpu.make_async_copy(
                table_hbm.at[pl.ds(0, 128)], rbuf.at[pl.ds(128, 128)], sem).wait()

        def fire_write(j, rbuf, sem):
            pltpu.async_copy(rbuf, out_hbm.at[pl.ds(j * _CH, _CH)], sem)

        def drain_write(rbuf, sem):
            pltpu.make_async_copy(
                out_hbm.at[pl.ds(0, _CH)], rbuf, sem).wait()

        fire_gather(j0, r0, gs0)

        def body(jj, carry):
            ja = j0 + 2 * jj
            fire_gather(ja + 1, r1, gs1)
            drain_gather(r0, gs0)
            fire_write(ja, r0, ws0)
            drain_gather(r1, gs1)
            fire_write(ja + 1, r1, ws1)
            drain_write(r0, ws0)

            @pl.when(jj < cpw // 2 - 1)
            def _():
                fire_gather(ja + 2, r0, gs0)

            drain_write(r1, ws1)
            return carry

        lax.fori_loop(0, cpw // 2, body, 0)

    return gather


def _make_gather_tiled(base_t, num_t):
    nchunks = num_t * _CPR
    cpw = nchunks // _NW
    span = (cpw - 1) // _CPR + 2
    nslab = 2

    @functools.partial(
        pl.kernel,
        out_type=jax.ShapeDtypeStruct((num_t * _B, _D), jnp.float32),
        mesh=_mesh,
        compiler_params=pltpu.CompilerParams(
            use_tc_tiling_on_sc=False, needs_layout_passes=False),
        scratch_types=[
            pltpu.VMEM((nslab, 32, 8, 128), jnp.int32),
            pltpu.VMEM((_CH, _D), jnp.float32),
            pltpu.VMEM((_CH, _D), jnp.float32),
            pltpu.SemaphoreType.DMA,
            pltpu.SemaphoreType.DMA,
            pltpu.SemaphoreType.DMA,
            pltpu.SemaphoreType.DMA,
        ],
    )
    def gather(tok_hbm, table_hbm, out_hbm, idx, r0, r1, gs0, gs1, ws0, ws1):
        w = lax.axis_index("s") * _NC + lax.axis_index("c")
        j0 = w * cpw
        s0 = jnp.minimum((base_t + j0 // _CPR) // 8, _DT // 8 - nslab)

        pltpu.sync_copy(tok_hbm.at[pl.ds(s0, nslab)], idx)

        def chunk_coords(j):
            t = base_t + j // _CPR
            tr = t // 8
            return tr - s0, (j - (j // _CPR) * _CPR) * 2, t - tr * 8

        def shift(i, carry):
            j = j0 + i // 16
            blk = i - (i // 16) * 16
            tch = blk // 8
            l0 = (blk - tch * 8) * 16
            ls, tc0, r = chunk_coords(j)
            ref = idx.at[ls, tc0 + tch, r]
            ref[pl.ds(l0, 16)] = ref[pl.ds(l0, 16)] + 1
            return carry
        lax.fori_loop(0, cpw * 16, shift, 0)

        def fire_gather(j, rbuf, sem):
            ls, tc0, r = chunk_coords(j)
            pltpu.async_copy(
                table_hbm.at[idx.at[ls, tc0, r]],
                rbuf.at[pl.ds(0, 128)], sem)
            pltpu.async_copy(
                table_hbm.at[idx.at[ls, tc0 + 1, r]],
                rbuf.at[pl.ds(128, 128)], sem)

        def drain_gather(rbuf, sem):
            pltpu.make_async_copy(
                table_hbm.at[pl.ds(0, 128)], rbuf.at[pl.ds(0, 128)], sem).wait()
            pltpu.make_async_copy(
                table_hbm.at[pl.ds(0, 128)], rbuf.at[pl.ds(128, 128)], sem).wait()

        def fire_write(j, rbuf, sem):
            pltpu.async_copy(rbuf, out_hbm.at[pl.ds(j * _CH, _CH)], sem)

        def drain_write(rbuf, sem):
            pltpu.make_async_copy(
                out_hbm.at[pl.ds(0, _CH)], rbuf, sem).wait()

        fire_gather(j0, r0, gs0)

        def body(jj, carry):
            ja = j0 + 2 * jj
            fire_gather(ja + 1, r1, gs1)
            drain_gather(r0, gs0)
            fire_write(ja, r0, ws0)
            drain_gather(r1, gs1)
            fire_write(ja + 1, r1, ws1)
            drain_write(r0, ws0)

            @pl.when(jj < cpw // 2 - 1)
            def _():
                fire_gather(ja + 2, r0, gs0)

            drain_write(r1, ws1)
            return carry

        lax.fori_loop(0, cpw // 2, body, 0)

    return gather


_gather_q = _make_gather(_QT, 0, _QT)
_gather_d0 = _make_gather_tiled(0, _DT // 2)
_gather_d1 = _make_gather_tiled(_DT // 2, _DT // 2)


def kernel(query_tok, doc_tok, table):
    q2 = query_tok.T.astype(jnp.int32)
    d4 = doc_tok.reshape(32, 128, 25, 8).transpose(2, 0, 3, 1).astype(jnp.int32)
    qf = _gather_q(q2, table)
    d0f = _gather_d0(d4, table)
    d1f = _gather_d1(d4, table)

    def to_bt(f, num_t):
        z = f.reshape(num_t, 2048, 2, _D)
        return z.transpose(1, 2, 0, 3).reshape(_B, num_t, _D)

    q_emb = to_bt(qf, _QT)
    d_emb = jnp.concatenate(
        [to_bt(d0f, _DT // 2), to_bt(d1f, _DT // 2)], axis=1)
    return (q_emb, d_emb)


# speedup vs baseline: 1.6398x; 1.6398x over previous
import functools

import jax
import jax.numpy as jnp
from jax import lax
from jax.experimental import pallas as pl
from jax.experimental.pallas import tpu as pltpu
from jax.experimental.pallas import tpu_sc as plsc

_D = 64
_B = 4096
_QT = 20
_DT = 200
_NC = 2
_NS = 16
_NW = _NC * _NS
_CH = 256
_CPR = _B // _CH

_mesh = plsc.VectorSubcoreMesh(
    core_axis_name="c", subcore_axis_name="s", num_cores=_NC, num_subcores=_NS
)


def _make_gather(total_t, base_t, num_t):
    nchunks = num_t * _CPR
    cpw = nchunks // _NW
    span = (cpw - 1) // _CPR + 2
    span = min(span, num_t)

    @functools.partial(
        pl.kernel,
        out_type=jax.ShapeDtypeStruct((num_t * _B, _D), jnp.float32),
        mesh=_mesh,
        compiler_params=pltpu.CompilerParams(
            use_tc_tiling_on_sc=False, needs_layout_passes=False),
        scratch_types=[
            pltpu.VMEM((span, _B), jnp.int32),
            pltpu.VMEM((_CH, _D), jnp.float32),
            pltpu.VMEM((_CH, _D), jnp.float32),
            pltpu.SemaphoreType.DMA,
            pltpu.SemaphoreType.DMA,
            pltpu.SemaphoreType.DMA,
            pltpu.SemaphoreType.DMA,
        ],
    )
    def gather(tok_hbm, table_hbm, out_hbm, idx, r0, r1, gs0, gs1, ws0, ws1):
        w = lax.axis_index("s") * _NC + lax.axis_index("c")
        j0 = w * cpw
        t0 = jnp.minimum(j0 // _CPR, num_t - span)

        pltpu.sync_copy(tok_hbm.at[pl.ds(base_t + t0, span)], idx)

        def shift(i, carry):
            r = i // (_B // 16)
            o = (i - r * (_B // 16)) * 16
            idx[r, pl.ds(o, 16)] = idx[r, pl.ds(o, 16)] + 1
            return carry
        lax.fori_loop(0, span * (_B // 16), shift, 0)

        def fire_gather(j, rbuf, sem):
            t = j // _CPR
            off = (j - t * _CPR) * _CH
            lr = t - t0
            pltpu.async_copy(
                table_hbm.at[idx.at[lr, pl.ds(off, 128)]],
                rbuf.at[pl.ds(0, 128)], sem)
            pltpu.async_copy(
                table_hbm.at[idx.at[lr, pl.ds(off + 128, 128)]],
                rbuf.at[pl.ds(128, 128)], sem)

        def drain_gather(rbuf, sem):
            pltpu.make_async_copy(
                table_hbm.at[pl.ds(0, 128)], rbuf.at[pl.ds(0, 128)], sem).wait()
            pltpu.make_async_copy(
                table_hbm.at[pl.ds(0, 128)], rbuf.at[pl.ds(128, 128)], sem).wait()

        def fire_write(j, rbuf, sem):
            pltpu.async_copy(rbuf, out_hbm.at[pl.ds(j * _CH, _CH)], sem)

        def drain_write(rbuf, sem):
            pltpu.make_async_copy(
                out_hbm.at[pl.ds(0, _CH)], rbuf, sem).wait()

        fire_gather(j0, r0, gs0)

        def body(jj, carry):
            ja = j0 + 2 * jj
            fire_gather(ja + 1, r1, gs1)
            drain_gather(r0, gs0)
            fire_write(ja, r0, ws0)
            drain_gather(r1, gs1)
            fire_write(ja + 1, r1, ws1)
            drain_write(r0, ws0)

            @pl.when(jj < cpw // 2 - 1)
            def _():
                fire_gather(ja + 2, r0, gs0)

            drain_write(r1, ws1)
            return carry

        lax.fori_loop(0, cpw // 2, body, 0)

    return gather


def _make_gather_tiled(base_t, num_t):
    nchunks = num_t * _CPR
    cpw = nchunks // _NW
    span = (cpw - 1) // _CPR + 2
    nslab = 2

    @functools.partial(
        pl.kernel,
        out_type=jax.ShapeDtypeStruct((num_t * _B, _D), jnp.float32),
        mesh=_mesh,
        compiler_params=pltpu.CompilerParams(
            use_tc_tiling_on_sc=False, needs_layout_passes=False),
        scratch_types=[
            pltpu.VMEM((nslab, 32, 8, 128), jnp.int32),
            pltpu.VMEM((_CH, _D), jnp.float32),
            pltpu.VMEM((_CH, _D), jnp.float32),
            pltpu.SemaphoreType.DMA,
            pltpu.SemaphoreType.DMA,
            pltpu.SemaphoreType.DMA,
            pltpu.SemaphoreType.DMA,
        ],
    )
    def gather(tok_hbm, table_hbm, out_hbm, idx, r0, r1, gs0, gs1, ws0, ws1):
        w = lax.axis_index("s") * _NC + lax.axis_index("c")
        j0 = w * cpw
        s0 = jnp.minimum((base_t + j0 // _CPR) // 8, _DT // 8 - nslab)

        pltpu.sync_copy(tok_hbm.at[pl.ds(s0, nslab)], idx)

        def chunk_coords(j):
            t = base_t + j // _CPR
            tr = t // 8
            return tr - s0, (j - (j // _CPR) * _CPR) * 2, t - tr * 8

        def shift(i, carry):
            j = j0 + i // 16
            blk = i - (i // 16) * 16
            tch = blk // 8
            l0 = (blk - tch * 8) * 16
            ls, tc0, r = chunk_coords(j)
            ref = idx.at[ls, tc0 + tch, r]
            ref[pl.ds(l0, 16)] = ref[pl.ds(l0, 16)] + 1
            return carry
        lax.fori_loop(0, cpw * 16, shift, 0)

        def fire_gather(j, rbuf, sem):
            ls, tc0, r = chunk_coords(j)
            pltpu.async_copy(
                table_hbm.at[idx.at[ls, tc0, r]],
                rbuf.at[pl.ds(0, 128)], sem)
            pltpu.async_copy(
                table_hbm.at[idx.at[ls, tc0 + 1, r]],
                rbuf.at[pl.ds(128, 128)], sem)

        def drain_gather(rbuf, sem):
            pltpu.make_async_copy(
                table_hbm.at[pl.ds(0, 128)], rbuf.at[pl.ds(0, 128)], sem).wait()
            pltpu.make_async_copy(
                table_hbm.at[pl.ds(0, 128)], rbuf.at[pl.ds(128, 128)], sem).wait()

        def fire_write(j, rbuf, sem):
            pltpu.async_copy(rbuf, out_hbm.at[pl.ds(j * _CH, _CH)], sem)

        def drain_write(rbuf, sem):
            pltpu.make_async_copy(
                out_hbm.at[pl.ds(0, _CH)], rbuf, sem).wait()

        fire_gather(j0, r0, gs0)

        def body(jj, carry):
            ja = j0 + 2 * jj
            fire_gather(ja + 1, r1, gs1)
            drain_gather(r0, gs0)
            fire_write(ja, r0, ws0)
            drain_gather(r1, gs1)
            fire_write(ja + 1, r1, ws1)
            drain_write(r0, ws0)

            @pl.when(jj < cpw // 2 - 1)
            def _():
                fire_gather(ja + 2, r0, gs0)

            drain_write(r1, ws1)
            return carry

        lax.fori_loop(0, cpw // 2, body, 0)

    return gather


_gather_q = _make_gather(_QT, 0, _QT)
_DSPLIT = 5
_DNT = _DT // _DSPLIT
_gather_d = [_make_gather_tiled(i * _DNT, _DNT) for i in range(_DSPLIT)]


def kernel(query_tok, doc_tok, table):
    q2 = query_tok.T.astype(jnp.int32)
    d4 = doc_tok.reshape(32, 128, 25, 8).transpose(2, 0, 3, 1).astype(jnp.int32)
    qf = _gather_q(q2, table)
    dfs = [g(d4, table) for g in _gather_d]

    q_emb = qf.reshape(_QT, _B, _D).transpose(1, 0, 2)
    d_emb = jnp.concatenate(
        [f.reshape(_DNT, _B, _D) for f in dfs], axis=0).transpose(1, 0, 2)
    return (q_emb, d_emb)
